# baseline (speedup 1.0000x reference)
import functools

import jax
import jax.numpy as jnp
from jax import lax
from jax.experimental import pallas as pl
from jax.experimental.pallas import tpu as pltpu
from jax.experimental.pallas import tpu_sc as plsc

_NW = 32
_CH = 128
_L = 16
_NTC = 7813


def kernel(indices, offsets, table):
    del offsets
    B = indices.shape[0]
    V, D = table.shape
    b_per_w = B // _NW
    n_ch = b_per_w // _CH
    p3 = (indices >> 1).reshape(_NW, n_ch, _CH).astype(jnp.int32)
    h3 = (indices & 1).reshape(_NW, n_ch, _CH).astype(jnp.int32)
    tableT = table.T

    n_full = _NTC - 1
    ragged_w = n_full % _NW
    kmax = (n_full + _NW - 1) // _NW

    mesh = plsc.VectorSubcoreMesh(core_axis_name="c", subcore_axis_name="s")

    @functools.partial(
        pl.kernel,
        mesh=mesh,
        compiler_params=pltpu.CompilerParams(needs_layout_passes=False),
        out_type=(
            jax.ShapeDtypeStruct((D, B), jnp.float32),
            jax.ShapeDtypeStruct((V // 2, 2 * D), jnp.float32),
        ),
        scratch_types=[
            pltpu.VMEM((n_ch, _CH), jnp.int32),
            pltpu.VMEM((n_ch, _CH), jnp.int32),
            pltpu.VMEM((D, 2 * D), jnp.float32),
            pltpu.VMEM((D, 2 * D), jnp.float32),
            pltpu.VMEM((D, 2 * D), jnp.float32),
            pltpu.VMEM((D, 2 * D), jnp.float32),
            pltpu.VMEM((_CH, 2 * D), jnp.float32),
            pltpu.VMEM((_CH, 2 * D), jnp.float32),
            pltpu.VMEM((D, _CH), jnp.float32),
            pltpu.VMEM((D, _CH), jnp.float32),
            pltpu.SemaphoreType.DMA,
            pltpu.SemaphoreType.DMA,
            pltpu.SemaphoreType.DMA,
            pltpu.SemaphoreType.DMA,
            pltpu.SemaphoreType.REGULAR,
        ],
    )
    def _fused(p_hbm, h_hbm, tableT_hbm, tailp_hbm, outT_hbm, scr_hbm,
               pv, hv, tbuf0, tbuf1, rbuf0, rbuf1, gbuf0, gbuf1,
               obuf0, obuf1, lsem, ssem, gsem, osem, bsem):
        wid = lax.axis_index("s") * 2 + lax.axis_index("c")
        base = wid * b_per_w
        pltpu.sync_copy(p_hbm.at[wid], pv)
        pltpu.sync_copy(h_hbm.at[wid], hv)

        iota = lax.iota(jnp.int32, _L)
        half = iota // 2
        parity64 = (iota % 2) * D
        rows_g = tuple(g * 8 + half for g in range(8))
        cols_g = tuple(g * _L + iota for g in range(8))

        def wait_load(buf):
            pltpu.make_async_copy(
                tableT_hbm.at[:, pl.ds(0, _CH)], buf, lsem).wait()

        def wait_store(buf):
            pltpu.make_async_copy(
                buf, scr_hbm.at[pl.ds(0, D), :], ssem).wait()

        def start_load(tc, buf):
            pltpu.async_copy(
                tableT_hbm.at[:, pl.ds(tc * _CH, _CH)], buf, lsem)

        def start_store(tc, buf):
            pltpu.async_copy(
                buf, scr_hbm.at[pl.ds(tc * D, D), :], ssem)

        def transpose_panel(tbuf, rbuf, ngr):
            def cbody(ci, carry):
                for u in range(2):
                    c = 2 * ci + u
                    cc = jnp.full((_L,), c, jnp.int32)
                    for g in range(ngr):
                        x = plsc.load_gather(tbuf, [cc, cols_g[g]])
                        m = parity64 + ((cc + cols_g[g]) & 63)
                        plsc.store_scatter(rbuf, [rows_g[g], m], x)
                return carry
            lax.fori_loop(0, D // 2, cbody, 0)

        @pl.when(wid < n_full)
        def _():
            start_load(wid, tbuf0)

        def p1body(kk, carry):
            for b, (tb, rb) in enumerate(((tbuf0, rbuf0), (tbuf1, rbuf1))):
                k = 2 * kk + b
                tc = k * _NW + wid
                nxt = tc + _NW

                @pl.when(tc < n_full)
                def _():
                    wait_load(tb)

                    @pl.when(nxt < n_full)
                    def _():
                        start_load(nxt, tbuf1 if b == 0 else tbuf0)

                    @pl.when(k >= 2)
                    def _():
                        wait_store(rb)
                    transpose_panel(tb, rb, 8)
                    start_store(tc, rb)
            return carry

        lax.fori_loop(0, (kmax + 1) // 2, p1body, 0)

        nvalid = lax.select(wid < (n_full - (kmax - 1) * _NW),
                            jnp.int32(kmax), jnp.int32(kmax - 1))

        def draino(i, carry):
            @pl.when(i < lax.min(nvalid, jnp.int32(2)))
            def _():
                wait_store(rbuf0)
            return carry

        lax.fori_loop(0, 2, draino, 0)

        @pl.when(wid == ragged_w)
        def _():
            pltpu.sync_copy(tailp_hbm, tbuf0.at[pl.ds(0, D // 2), :])

            def tcbody(c, carry):
                cc = jnp.full((_L,), c, jnp.int32)
                for g in range(4):
                    x = plsc.load_gather(
                        tbuf0, [rows_g[g], parity64 + c])
                    m = parity64 + ((cc + cols_g[g]) & 63)
                    plsc.store_scatter(rbuf0, [rows_g[g], m], x)
                return carry

            lax.fori_loop(0, D, tcbody, 0)
            pltpu.sync_copy(
                rbuf0.at[pl.ds(0, D // 2), :],
                scr_hbm.at[pl.ds(n_full * D, D // 2), :])

        plsc.subcore_barrier()
        pltpu.core_barrier(bsem, core_axis_name="c")

        def wait_gather(buf):
            pltpu.make_async_copy(
                scr_hbm.at[pl.ds(0, _CH)], buf, gsem).wait()

        def wait_ostore(buf):
            pltpu.make_async_copy(
                buf, outT_hbm.at[:, pl.ds(0, _CH)], osem).wait()

        pltpu.async_copy(scr_hbm.at[pv.at[0]], gbuf0, gsem)

        def select_chunk(j, gbuf, obuf):
            h64 = []
            rfull = []
            for g in range(8):
                h = hv[j, pl.ds(g * _L, _L)]
                p = pv[j, pl.ds(g * _L, _L)]
                h64.append(h * D)
                rfull.append(2 * p + h)

            def cbody(ci, carry):
                for u in range(2):
                    c = 2 * ci + u
                    cc = jnp.full((_L,), c, jnp.int32)
                    for g in range(8):
                        m = h64[g] + ((cc + rfull[g]) & 63)
                        x = plsc.load_gather(gbuf, [cols_g[g], m])
                        plsc.store_scatter(obuf, [cc, cols_g[g]], x)
                return carry

            lax.fori_loop(0, D // 2, cbody, 0)

        def p2body(jj, carry):
            for b, (gb, ob) in enumerate(((gbuf0, obuf0), (gbuf1, obuf1))):
                j = 2 * jj + b
                wait_gather(gb)

                @pl.when(j + 1 < n_ch)
                def _():
                    pltpu.async_copy(
                        scr_hbm.at[pv.at[j + 1]],
                        gbuf1 if b == 0 else gbuf0, gsem)

                @pl.when(j >= 2)
                def _():
                    wait_ostore(ob)
                select_chunk(j, gb, ob)
                pltpu.async_copy(
                    ob, outT_hbm.at[:, pl.ds(base + j * _CH, _CH)], osem)
            return carry

        lax.fori_loop(0, n_ch // 2, p2body, 0)
        wait_ostore(obuf0)
        wait_ostore(obuf1)

    tailp = table[(V // _CH) * _CH:].reshape(D // 2, 2 * D)
    outT, _ = _fused(p3, h3, tableT, tailp)
    return outT.T


# speedup vs baseline: 1.5894x; 1.5894x over previous
import functools

import jax
import jax.numpy as jnp
from jax import lax
from jax.experimental import pallas as pl
from jax.experimental.pallas import tpu as pltpu
from jax.experimental.pallas import tpu_sc as plsc

_NW = 32
_CH = 128
_L = 16
_NTC = 7813


def kernel(indices, offsets, table):
    del offsets
    B = indices.shape[0]
    V, D = table.shape
    b_per_w = B // _NW
    n_ch = b_per_w // _CH
    p3 = (indices >> 1).reshape(_NW, n_ch, _CH).astype(jnp.int32)
    h3 = (indices & 1).reshape(_NW, n_ch, _CH).astype(jnp.int32)
    tableT = table.T

    n_full = _NTC - 1
    ragged_w = n_full % _NW
    kmax = (n_full + _NW - 1) // _NW

    mesh = plsc.VectorSubcoreMesh(core_axis_name="c", subcore_axis_name="s")

    @functools.partial(
        pl.kernel,
        mesh=mesh,
        compiler_params=pltpu.CompilerParams(needs_layout_passes=False),
        out_type=(
            jax.ShapeDtypeStruct((D, B), jnp.float32),
            jax.ShapeDtypeStruct((V // 2, 2 * D), jnp.float32),
        ),
        scratch_types=[
            pltpu.VMEM((n_ch, _CH), jnp.int32),
            pltpu.VMEM((n_ch, _CH), jnp.int32),
            pltpu.VMEM((D, 2 * D), jnp.float32),
            pltpu.VMEM((D, 2 * D), jnp.float32),
            pltpu.VMEM((D, 2 * D), jnp.float32),
            pltpu.VMEM((D, 2 * D), jnp.float32),
            pltpu.VMEM((_CH, 2 * D), jnp.float32),
            pltpu.VMEM((_CH, 2 * D), jnp.float32),
            pltpu.VMEM((D, _CH), jnp.float32),
            pltpu.VMEM((D, _CH), jnp.float32),
            pltpu.SemaphoreType.DMA,
            pltpu.SemaphoreType.DMA,
            pltpu.SemaphoreType.DMA,
            pltpu.SemaphoreType.DMA,
            pltpu.SemaphoreType.REGULAR,
        ],
    )
    def _fused(p_hbm, h_hbm, tableT_hbm, tailp_hbm, outT_hbm, scr_hbm,
               pv, hv, tbuf0, tbuf1, rbuf0, rbuf1, gbuf0, gbuf1,
               obuf0, obuf1, lsem, ssem, gsem, osem, bsem):
        wid = lax.axis_index("s") * 2 + lax.axis_index("c")
        base = wid * b_per_w
        pltpu.sync_copy(p_hbm.at[wid], pv)
        pltpu.sync_copy(h_hbm.at[wid], hv)

        iota = lax.iota(jnp.int32, _L)
        half = iota // 2
        parity64 = (iota % 2) * D
        rows_g = tuple(g * 8 + half for g in range(8))
        cols_g = tuple(g * _L + iota for g in range(8))

        def wait_load(buf):
            pltpu.make_async_copy(
                tableT_hbm.at[:, pl.ds(0, _CH)], buf, lsem).wait()

        def wait_store(buf):
            pltpu.make_async_copy(
                buf, scr_hbm.at[pl.ds(0, D), :], ssem).wait()

        def start_load(tc, buf):
            pltpu.async_copy(
                tableT_hbm.at[:, pl.ds(tc * _CH, _CH)], buf, lsem)

        def start_store(tc, buf):
            pltpu.async_copy(
                buf, scr_hbm.at[pl.ds(tc * D, D), :], ssem)

        def transpose_panel(tbuf, rbuf, ngr):
            def cbody(ci, carry):
                for u in range(2):
                    c = 2 * ci + u
                    cc = jnp.full((_L,), c, jnp.int32)
                    ms = [parity64 + ((cc + cols_g[g]) & 63)
                          for g in range(ngr)]
                    xs = [plsc.load_gather(tbuf, [cc, cols_g[g]])
                          for g in range(ngr)]
                    for g in range(ngr):
                        plsc.store_scatter(rbuf, [rows_g[g], ms[g]], xs[g])
                return carry
            lax.fori_loop(0, D // 2, cbody, 0)

        @pl.when(wid < n_full)
        def _():
            start_load(wid, tbuf0)

        def p1body(kk, carry):
            for b, (tb, rb) in enumerate(((tbuf0, rbuf0), (tbuf1, rbuf1))):
                k = 2 * kk + b
                tc = k * _NW + wid
                nxt = tc + _NW

                @pl.when(tc < n_full)
                def _():
                    wait_load(tb)

                    @pl.when(nxt < n_full)
                    def _():
                        start_load(nxt, tbuf1 if b == 0 else tbuf0)

                    @pl.when(k >= 2)
                    def _():
                        wait_store(rb)
                    transpose_panel(tb, rb, 8)
                    start_store(tc, rb)
            return carry

        lax.fori_loop(0, (kmax + 1) // 2, p1body, 0)

        nvalid = lax.select(wid < (n_full - (kmax - 1) * _NW),
                            jnp.int32(kmax), jnp.int32(kmax - 1))

        def draino(i, carry):
            @pl.when(i < lax.min(nvalid, jnp.int32(2)))
            def _():
                wait_store(rbuf0)
            return carry

        lax.fori_loop(0, 2, draino, 0)

        @pl.when(wid == ragged_w)
        def _():
            pltpu.sync_copy(tailp_hbm, tbuf0.at[pl.ds(0, D // 2), :])

            def tcbody(c, carry):
                cc = jnp.full((_L,), c, jnp.int32)
                for g in range(4):
                    x = plsc.load_gather(
                        tbuf0, [rows_g[g], parity64 + c])
                    m = parity64 + ((cc + cols_g[g]) & 63)
                    plsc.store_scatter(rbuf0, [rows_g[g], m], x)
                return carry

            lax.fori_loop(0, D, tcbody, 0)
            pltpu.sync_copy(
                rbuf0.at[pl.ds(0, D // 2), :],
                scr_hbm.at[pl.ds(n_full * D, D // 2), :])

        plsc.subcore_barrier()
        pltpu.core_barrier(bsem, core_axis_name="c")

        def wait_gather(buf):
            pltpu.make_async_copy(
                scr_hbm.at[pl.ds(0, _CH)], buf, gsem).wait()

        def wait_ostore(buf):
            pltpu.make_async_copy(
                buf, outT_hbm.at[:, pl.ds(0, _CH)], osem).wait()

        pltpu.async_copy(scr_hbm.at[pv.at[0]], gbuf0, gsem)

        def select_chunk(j, gbuf, obuf):
            h64 = []
            rfull = []
            for g in range(8):
                h = hv[j, pl.ds(g * _L, _L)]
                p = pv[j, pl.ds(g * _L, _L)]
                h64.append(h * D)
                rfull.append(2 * p + h)

            def cbody(ci, carry):
                for u in range(2):
                    c = 2 * ci + u
                    cc = jnp.full((_L,), c, jnp.int32)
                    xs = [plsc.load_gather(
                        gbuf, [cols_g[g], h64[g] + ((cc + rfull[g]) & 63)])
                        for g in range(8)]
                    for g in range(8):
                        plsc.store_scatter(obuf, [cc, cols_g[g]], xs[g])
                return carry

            lax.fori_loop(0, D // 2, cbody, 0)

        def p2body(jj, carry):
            for b, (gb, ob) in enumerate(((gbuf0, obuf0), (gbuf1, obuf1))):
                j = 2 * jj + b
                wait_gather(gb)

                @pl.when(j + 1 < n_ch)
                def _():
                    pltpu.async_copy(
                        scr_hbm.at[pv.at[j + 1]],
                        gbuf1 if b == 0 else gbuf0, gsem)

                @pl.when(j >= 2)
                def _():
                    wait_ostore(ob)
                select_chunk(j, gb, ob)
                pltpu.async_copy(
                    ob, outT_hbm.at[:, pl.ds(base + j * _CH, _CH)], osem)
            return carry

        lax.fori_loop(0, n_ch // 2, p2body, 0)
        wait_ostore(obuf0)
        wait_ostore(obuf1)

    tailp = table[(V // _CH) * _CH:].reshape(D // 2, 2 * D)
    outT, _ = _fused(p3, h3, tableT, tailp)
    return outT.T


# speedup vs baseline: 1.5896x; 1.0001x over previous
import functools

import jax
import jax.numpy as jnp
from jax import lax
from jax.experimental import pallas as pl
from jax.experimental.pallas import tpu as pltpu
from jax.experimental.pallas import tpu_sc as plsc

_NW = 32
_CH = 128
_L = 16
_NTC = 7813


def kernel(indices, offsets, table):
    del offsets
    B = indices.shape[0]
    V, D = table.shape
    b_per_w = B // _NW
    n_ch = b_per_w // _CH
    p3 = (indices >> 1).reshape(_NW, n_ch, _CH).astype(jnp.int32)
    h3 = (indices & 1).reshape(_NW, n_ch, _CH).astype(jnp.int32)
    tableT = table.T

    n_full = _NTC - 1
    ragged_w = n_full % _NW
    kmax = (n_full + _NW - 1) // _NW

    mesh = plsc.VectorSubcoreMesh(core_axis_name="c", subcore_axis_name="s")

    @functools.partial(
        pl.kernel,
        mesh=mesh,
        compiler_params=pltpu.CompilerParams(needs_layout_passes=False),
        out_type=(
            jax.ShapeDtypeStruct((D, B), jnp.float32),
            jax.ShapeDtypeStruct((V // 2, 2 * D), jnp.float32),
        ),
        scratch_types=[
            pltpu.VMEM((n_ch, _CH), jnp.int32),
            pltpu.VMEM((n_ch, _CH), jnp.int32),
            pltpu.VMEM((D, 2 * D), jnp.float32),
            pltpu.VMEM((D, 2 * D), jnp.float32),
            pltpu.VMEM((D, 2 * D), jnp.float32),
            pltpu.VMEM((D, 2 * D), jnp.float32),
            pltpu.VMEM((_CH, 2 * D), jnp.float32),
            pltpu.VMEM((_CH, 2 * D), jnp.float32),
            pltpu.VMEM((D, _CH), jnp.float32),
            pltpu.VMEM((D, _CH), jnp.float32),
            pltpu.SemaphoreType.DMA,
            pltpu.SemaphoreType.DMA,
            pltpu.SemaphoreType.DMA,
            pltpu.SemaphoreType.DMA,
            pltpu.SemaphoreType.REGULAR,
        ],
    )
    def _fused(p_hbm, h_hbm, tableT_hbm, tailp_hbm, outT_hbm, scr_hbm,
               pv, hv, tbuf0, tbuf1, rbuf0, rbuf1, gbuf0, gbuf1,
               obuf0, obuf1, lsem, ssem, gsem, osem, bsem):
        wid = lax.axis_index("s") * 2 + lax.axis_index("c")
        base = wid * b_per_w
        pltpu.sync_copy(p_hbm.at[wid], pv)
        pltpu.sync_copy(h_hbm.at[wid], hv)

        iota = lax.iota(jnp.int32, _L)
        half = iota // 2
        parity64 = (iota % 2) * D
        rows_g = tuple(g * 8 + half for g in range(8))
        cols_g = tuple(g * _L + iota for g in range(8))

        def wait_load(buf):
            pltpu.make_async_copy(
                tableT_hbm.at[:, pl.ds(0, _CH)], buf, lsem).wait()

        def wait_store(buf):
            pltpu.make_async_copy(
                buf, scr_hbm.at[pl.ds(0, D), :], ssem).wait()

        def start_load(tc, buf):
            pltpu.async_copy(
                tableT_hbm.at[:, pl.ds(tc * _CH, _CH)], buf, lsem)

        def start_store(tc, buf):
            pltpu.async_copy(
                buf, scr_hbm.at[pl.ds(tc * D, D), :], ssem)

        def transpose_panel(tbuf, rbuf, ngr):
            def cbody(ci, carry):
                ccs = [jnp.full((_L,), 2 * ci + u, jnp.int32)
                       for u in range(2)]
                ms = [[parity64 + ((cc + cols_g[g]) & 63)
                       for g in range(ngr)] for cc in ccs]
                xs = [[plsc.load_gather(tbuf, [cc, cols_g[g]])
                       for g in range(ngr)] for cc in ccs]
                for u in range(2):
                    for g in range(ngr):
                        plsc.store_scatter(
                            rbuf, [rows_g[g], ms[u][g]], xs[u][g])
                return carry
            lax.fori_loop(0, D // 2, cbody, 0)

        @pl.when(wid < n_full)
        def _():
            start_load(wid, tbuf0)

        def p1body(kk, carry):
            for b, (tb, rb) in enumerate(((tbuf0, rbuf0), (tbuf1, rbuf1))):
                k = 2 * kk + b
                tc = k * _NW + wid
                nxt = tc + _NW

                @pl.when(tc < n_full)
                def _():
                    wait_load(tb)

                    @pl.when(nxt < n_full)
                    def _():
                        start_load(nxt, tbuf1 if b == 0 else tbuf0)

                    @pl.when(k >= 2)
                    def _():
                        wait_store(rb)
                    transpose_panel(tb, rb, 8)
                    start_store(tc, rb)
            return carry

        lax.fori_loop(0, (kmax + 1) // 2, p1body, 0)

        nvalid = lax.select(wid < (n_full - (kmax - 1) * _NW),
                            jnp.int32(kmax), jnp.int32(kmax - 1))

        def draino(i, carry):
            @pl.when(i < lax.min(nvalid, jnp.int32(2)))
            def _():
                wait_store(rbuf0)
            return carry

        lax.fori_loop(0, 2, draino, 0)

        @pl.when(wid == ragged_w)
        def _():
            pltpu.sync_copy(tailp_hbm, tbuf0.at[pl.ds(0, D // 2), :])

            def tcbody(c, carry):
                cc = jnp.full((_L,), c, jnp.int32)
                for g in range(4):
                    x = plsc.load_gather(
                        tbuf0, [rows_g[g], parity64 + c])
                    m = parity64 + ((cc + cols_g[g]) & 63)
                    plsc.store_scatter(rbuf0, [rows_g[g], m], x)
                return carry

            lax.fori_loop(0, D, tcbody, 0)
            pltpu.sync_copy(
                rbuf0.at[pl.ds(0, D // 2), :],
                scr_hbm.at[pl.ds(n_full * D, D // 2), :])

        plsc.subcore_barrier()
        pltpu.core_barrier(bsem, core_axis_name="c")

        def wait_gather(buf):
            pltpu.make_async_copy(
                scr_hbm.at[pl.ds(0, _CH)], buf, gsem).wait()

        def wait_ostore(buf):
            pltpu.make_async_copy(
                buf, outT_hbm.at[:, pl.ds(0, _CH)], osem).wait()

        pltpu.async_copy(scr_hbm.at[pv.at[0]], gbuf0, gsem)

        def select_chunk(j, gbuf, obuf):
            h64 = []
            rfull = []
            for g in range(8):
                h = hv[j, pl.ds(g * _L, _L)]
                p = pv[j, pl.ds(g * _L, _L)]
                h64.append(h * D)
                rfull.append(2 * p + h)

            def cbody(ci, carry):
                ccs = [jnp.full((_L,), 2 * ci + u, jnp.int32)
                       for u in range(2)]
                xs = [[plsc.load_gather(
                    gbuf, [cols_g[g], h64[g] + ((cc + rfull[g]) & 63)])
                    for g in range(8)] for cc in ccs]
                for u in range(2):
                    for g in range(8):
                        plsc.store_scatter(obuf, [ccs[u], cols_g[g]], xs[u][g])
                return carry

            lax.fori_loop(0, D // 2, cbody, 0)

        def p2body(jj, carry):
            for b, (gb, ob) in enumerate(((gbuf0, obuf0), (gbuf1, obuf1))):
                j = 2 * jj + b
                wait_gather(gb)

                @pl.when(j + 1 < n_ch)
                def _():
                    pltpu.async_copy(
                        scr_hbm.at[pv.at[j + 1]],
                        gbuf1 if b == 0 else gbuf0, gsem)

                @pl.when(j >= 2)
                def _():
                    wait_ostore(ob)
                select_chunk(j, gb, ob)
                pltpu.async_copy(
                    ob, outT_hbm.at[:, pl.ds(base + j * _CH, _CH)], osem)
            return carry

        lax.fori_loop(0, n_ch // 2, p2body, 0)
        wait_ostore(obuf0)
        wait_ostore(obuf1)

    tailp = table[(V // _CH) * _CH:].reshape(D // 2, 2 * D)
    outT, _ = _fused(p3, h3, tableT, tailp)
    return outT.T
